# baseline (speedup 1.0000x reference)
import jax
import jax.numpy as jnp
from jax import lax
from jax.experimental import pallas as pl
from jax.experimental.pallas import tpu as pltpu
from jax.experimental.pallas import tpu_sc as plsc

D = 768
L = 16
NB = 16
NS_ = (1024, 256, 64)
OFF = (0, 1024, 1280)
NN = (32, 8, 2)
BIAS_OFF = (0, 32, 40)
RTOT = 1344
G = 2
UNROLL = 4


def _add_group(buf, bias, bias_r0):
    for r in range(G):
        @plsc.parallel_loop(0, D, step=L, unroll=UNROLL)
        def _(v, r=r):
            x = bias[bias_r0 + r, pl.ds(v, L)]
            for b in range(NB):
                plsc.addupdate(buf.at[b, r, pl.ds(v, L)], x)


def _sc_body(f0, f1, f2, sc_emb, patch, out, bias, srow, work,
             sin0, sin1, sout0, sout1):
    wid = lax.axis_index("subcore") * 2 + lax.axis_index("core")
    pltpu.sync_copy(sc_emb, srow)
    for i in range(3):
        n0 = wid * NN[i]
        pltpu.sync_copy(patch.at[pl.ds(2048 * i + n0, NN[i])],
                        bias.at[pl.ds(BIAS_OFF[i], NN[i])])
    for i in range(3):
        @pl.loop(0, NN[i])
        def _(r, i=i):
            @plsc.parallel_loop(0, D, step=L, unroll=UNROLL)
            def _(v, r=r, i=i):
                plsc.addupdate(bias.at[BIAS_OFF[i] + r, pl.ds(v, L)],
                               srow[i, pl.ds(v, L)])

    for i, fref in enumerate((f0, f1, f2)):
        nn = NN[i]
        n0 = wid * nn
        ngrp = nn // G

        def fsl(g, i=i, n0=n0, fref=fref):
            return fref.at[pl.ds(0, NB), pl.ds(n0 + g * G, G)]

        def osl(g, i=i, n0=n0):
            return out.at[pl.ds(0, NB), pl.ds(OFF[i] + n0 + g * G, G)]

        buf0 = work.at[0]
        buf1 = work.at[1]

        @pl.loop(0, ngrp, step=2)
        def _(g, i=i, fsl=fsl, osl=osl, buf0=buf0, buf1=buf1):
            @pl.when(g > 0)
            def _():
                pltpu.make_async_copy(buf0, osl(g - 2), sout0).wait()
                pltpu.make_async_copy(buf1, osl(g - 1), sout1).wait()
            pltpu.make_async_copy(fsl(g), buf0, sin0).start()
            pltpu.make_async_copy(fsl(g + 1), buf1, sin1).start()
            pltpu.make_async_copy(fsl(g), buf0, sin0).wait()
            _add_group(buf0, bias, BIAS_OFF[i] + g * G)
            pltpu.make_async_copy(buf0, osl(g), sout0).start()
            pltpu.make_async_copy(fsl(g + 1), buf1, sin1).wait()
            _add_group(buf1, bias, BIAS_OFF[i] + (g + 1) * G)
            pltpu.make_async_copy(buf1, osl(g + 1), sout1).start()

        pltpu.make_async_copy(buf0, osl(ngrp - 2), sout0).wait()
        pltpu.make_async_copy(buf1, osl(ngrp - 1), sout1).wait()


def kernel(features_per_scale_0, features_per_scale_1, features_per_scale_2,
           scale_embeddings, patch_embeddings):
    patch = patch_embeddings.reshape(3 * 2048, D)

    mesh = plsc.VectorSubcoreMesh(core_axis_name="core",
                                  subcore_axis_name="subcore")
    run = pl.kernel(
        _sc_body,
        out_type=jax.ShapeDtypeStruct((NB, RTOT, D), jnp.float32),
        mesh=mesh,
        scratch_types=[
            pltpu.VMEM((42, D), jnp.float32),
            pltpu.VMEM((3, D), jnp.float32),
            pltpu.VMEM((2, NB, G, D), jnp.float32),
            pltpu.SemaphoreType.DMA,
            pltpu.SemaphoreType.DMA,
            pltpu.SemaphoreType.DMA,
            pltpu.SemaphoreType.DMA,
        ],
    )
    return run(features_per_scale_0, features_per_scale_1, features_per_scale_2,
               scale_embeddings, patch)


# speedup vs baseline: 1.3909x; 1.3909x over previous
import jax
import jax.numpy as jnp
from jax import lax
from jax.experimental import pallas as pl
from jax.experimental.pallas import tpu as pltpu
from jax.experimental.pallas import tpu_sc as plsc

D = 768
L = 16
NB = 16
NS_ = (1024, 256, 64)
OFF = (0, 1024, 1280)
NN = (32, 8, 2)
BIAS_OFF = (0, 32, 40)
RTOT = 1344
G = 2
NBUF = 3
UNROLL = 4


def _add_group(buf, bias, bias_r0):
    for r in range(G):
        @plsc.parallel_loop(0, D, step=L, unroll=UNROLL)
        def _(v, r=r):
            x = bias[bias_r0 + r, pl.ds(v, L)]
            for b in range(NB):
                plsc.addupdate(buf.at[b, r, pl.ds(v, L)], x)


def _sc_body(f0, f1, f2, sc_emb, patch, out, bias, srow, work,
             *sems):
    sin = sems[:NBUF]
    sout = sems[NBUF:]
    wid = lax.axis_index("subcore") * 2 + lax.axis_index("core")
    pltpu.sync_copy(sc_emb, srow)
    for i in range(3):
        n0 = wid * NN[i]
        pltpu.sync_copy(patch.at[pl.ds(2048 * i + n0, NN[i])],
                        bias.at[pl.ds(BIAS_OFF[i], NN[i])])
    for i in range(3):
        @pl.loop(0, NN[i])
        def _(r, i=i):
            @plsc.parallel_loop(0, D, step=L, unroll=UNROLL)
            def _(v, r=r, i=i):
                plsc.addupdate(bias.at[BIAS_OFF[i] + r, pl.ds(v, L)],
                               srow[i, pl.ds(v, L)])

    bufs = tuple(work.at[k] for k in range(NBUF))

    for i, fref in enumerate((f0, f1, f2)):
        nn = NN[i]
        n0 = wid * nn
        ngrp = nn // G
        nloop = ((ngrp - 1) // 3) * 3

        def fsl(g, i=i, n0=n0, fref=fref):
            return fref.at[pl.ds(0, NB), pl.ds(n0 + g * G, G)]

        def osl(g, i=i, n0=n0):
            return out.at[pl.ds(0, NB), pl.ds(OFF[i] + n0 + g * G, G)]

        for t in range(min(2, ngrp)):
            pltpu.make_async_copy(fsl(t), bufs[t % 3], sin[t % 3]).start()

        if nloop > 0:
            @pl.loop(0, nloop, step=3)
            def _(g, i=i, ngrp=ngrp, fsl=fsl, osl=osl):
                for k in range(3):
                    X = bufs[k]
                    pltpu.make_async_copy(fsl(g + k), X, sin[k]).wait()
                    _add_group(X, bias, BIAS_OFF[i] + (g + k) * G)
                    pltpu.make_async_copy(X, osl(g + k), sout[k]).start()
                    kp = (k + 2) % 3
                    Y = bufs[kp]

                    @pl.when(g + k + 2 < ngrp)
                    def _(g=g, k=k, kp=kp, Y=Y, fsl=fsl, osl=osl):
                        @pl.when(g + k >= 1)
                        def _():
                            pltpu.make_async_copy(Y, osl(g + k - 1),
                                                  sout[kp]).wait()
                        pltpu.make_async_copy(fsl(g + k + 2), Y,
                                              sin[kp]).start()

        for t in range(nloop, ngrp):
            X = bufs[t % 3]
            pltpu.make_async_copy(fsl(t), X, sin[t % 3]).wait()
            _add_group(X, bias, BIAS_OFF[i] + t * G)
            pltpu.make_async_copy(X, osl(t), sout[t % 3]).start()

        for t in range(max(0, ngrp - 3), ngrp):
            pltpu.make_async_copy(bufs[t % 3], osl(t), sout[t % 3]).wait()


def kernel(features_per_scale_0, features_per_scale_1, features_per_scale_2,
           scale_embeddings, patch_embeddings):
    patch = patch_embeddings.reshape(3 * 2048, D)

    mesh = plsc.VectorSubcoreMesh(core_axis_name="core",
                                  subcore_axis_name="subcore")
    run = pl.kernel(
        _sc_body,
        out_type=jax.ShapeDtypeStruct((NB, RTOT, D), jnp.float32),
        mesh=mesh,
        scratch_types=[
            pltpu.VMEM((42, D), jnp.float32),
            pltpu.VMEM((3, D), jnp.float32),
            pltpu.VMEM((NBUF, NB, G, D), jnp.float32),
        ] + [pltpu.SemaphoreType.DMA] * (2 * NBUF),
    )
    return run(features_per_scale_0, features_per_scale_1, features_per_scale_2,
               scale_embeddings, patch)


# speedup vs baseline: 1.4068x; 1.0114x over previous
import jax
import jax.numpy as jnp
from jax import lax
from jax.experimental import pallas as pl
from jax.experimental.pallas import tpu as pltpu
from jax.experimental.pallas import tpu_sc as plsc

D = 768
L = 16
NB = 16
NS_ = (1024, 256, 64)
OFF = (0, 1024, 1280)
NN = (32, 8, 2)
BIAS_OFF = (0, 32, 40)
RTOT = 1344
G = 2
NBUF = 3
UNROLL = 4


def _add_group(buf, bias, bias_r0):
    for r in range(G):
        @plsc.parallel_loop(0, D, step=L, unroll=UNROLL)
        def _(v, r=r):
            x = bias[bias_r0 + r, pl.ds(v, L)]
            for b in range(NB):
                plsc.addupdate(buf.at[b, r, pl.ds(v, L)], x)


def _sc_body(f0, f1, f2, sc_emb, patch, out, bias, srow, work,
             *sems):
    sin = sems[:NBUF]
    sout = sems[NBUF:]
    wid = lax.axis_index("subcore") * 2 + lax.axis_index("core")
    bufs = tuple(work.at[k] for k in range(NBUF))

    for t in range(2):
        pltpu.make_async_copy(
            f0.at[pl.ds(0, NB), pl.ds(wid * NN[0] + t * G, G)],
            bufs[t], sin[t]).start()

    pltpu.sync_copy(sc_emb, srow)
    for i in range(3):
        n0 = wid * NN[i]
        pltpu.sync_copy(patch.at[pl.ds(2048 * i + n0, NN[i])],
                        bias.at[pl.ds(BIAS_OFF[i], NN[i])])
    for i in range(3):
        @pl.loop(0, NN[i])
        def _(r, i=i):
            @plsc.parallel_loop(0, D, step=L, unroll=UNROLL)
            def _(v, r=r, i=i):
                plsc.addupdate(bias.at[BIAS_OFF[i] + r, pl.ds(v, L)],
                               srow[i, pl.ds(v, L)])

    for i, fref in enumerate((f0, f1, f2)):
        nn = NN[i]
        n0 = wid * nn
        ngrp = nn // G
        nloop = ((ngrp - 1) // 3) * 3

        def fsl(g, i=i, n0=n0, fref=fref):
            return fref.at[pl.ds(0, NB), pl.ds(n0 + g * G, G)]

        def osl(g, i=i, n0=n0):
            return out.at[pl.ds(0, NB), pl.ds(OFF[i] + n0 + g * G, G)]

        if i > 0:
            for t in range(min(2, ngrp)):
                pltpu.make_async_copy(fsl(t), bufs[t % 3], sin[t % 3]).start()

        if nloop > 0:
            @pl.loop(0, nloop, step=3)
            def _(g, i=i, ngrp=ngrp, fsl=fsl, osl=osl):
                for k in range(3):
                    X = bufs[k]
                    pltpu.make_async_copy(fsl(g + k), X, sin[k]).wait()
                    _add_group(X, bias, BIAS_OFF[i] + (g + k) * G)
                    pltpu.make_async_copy(X, osl(g + k), sout[k]).start()
                    kp = (k + 2) % 3
                    Y = bufs[kp]

                    @pl.when(g + k + 2 < ngrp)
                    def _(g=g, k=k, kp=kp, Y=Y, fsl=fsl, osl=osl):
                        @pl.when(g + k >= 1)
                        def _():
                            pltpu.make_async_copy(Y, osl(g + k - 1),
                                                  sout[kp]).wait()
                        pltpu.make_async_copy(fsl(g + k + 2), Y,
                                              sin[kp]).start()

        for t in range(nloop, ngrp):
            X = bufs[t % 3]
            pltpu.make_async_copy(fsl(t), X, sin[t % 3]).wait()
            _add_group(X, bias, BIAS_OFF[i] + t * G)
            pltpu.make_async_copy(X, osl(t), sout[t % 3]).start()

        for t in range(max(0, ngrp - 3), ngrp):
            pltpu.make_async_copy(bufs[t % 3], osl(t), sout[t % 3]).wait()


def kernel(features_per_scale_0, features_per_scale_1, features_per_scale_2,
           scale_embeddings, patch_embeddings):
    patch = patch_embeddings.reshape(3 * 2048, D)

    mesh = plsc.VectorSubcoreMesh(core_axis_name="core",
                                  subcore_axis_name="subcore")
    run = pl.kernel(
        _sc_body,
        out_type=jax.ShapeDtypeStruct((NB, RTOT, D), jnp.float32),
        mesh=mesh,
        scratch_types=[
            pltpu.VMEM((42, D), jnp.float32),
            pltpu.VMEM((3, D), jnp.float32),
            pltpu.VMEM((NBUF, NB, G, D), jnp.float32),
        ] + [pltpu.SemaphoreType.DMA] * (2 * NBUF),
    )
    return run(features_per_scale_0, features_per_scale_1, features_per_scale_2,
               scale_embeddings, patch)
